# baseline (speedup 1.0000x reference)
import functools

import jax
import jax.numpy as jnp
from jax import lax
from jax.experimental import pallas as pl
from jax.experimental.pallas import tpu as pltpu, tpu_sc as plsc


def _zero_scalar_kernel(o_ref):
    o_ref[...] = jnp.zeros((1, 1), jnp.float32)


def _sc_zero():
    mesh = plsc.VectorSubcoreMesh(core_axis_name="c", subcore_axis_name="s")

    @functools.partial(
        pl.kernel,
        mesh=mesh,
        out_type=jax.ShapeDtypeStruct((16,), jnp.float32),
        scratch_types=[pltpu.VMEM((16,), jnp.float32)],
    )
    def k(out_hbm, v):
        wid = lax.axis_index("s") * 2 + lax.axis_index("c")

        @pl.when(wid == 0)
        def _():
            v[...] = jnp.zeros((16,), jnp.float32)
            pltpu.sync_copy(v, out_hbm)

    return k()


def _make_softplus_sum_kernel(scale):
    def _softplus_sum_kernel(x_ref, o_ref):
        @pl.when(pl.program_id(0) == 0)
        def _init():
            o_ref[...] = jnp.zeros((1, 1), jnp.float32)

        x = x_ref[...]
        sp = jnp.maximum(x, 0.0) + jnp.log1p(jnp.exp(-jnp.abs(x)))
        o_ref[...] += (jnp.sum(sp) * scale).reshape(1, 1)

    return _softplus_sum_kernel


def kernel(predictions, targets):
    B, A, H, W = predictions.shape
    n_targets = targets.shape[0]
    total_elems = B * A * H * W
    obj_mask_any = (n_targets > 0) and (total_elems > 0)
    noobj_mask_any = (total_elems > 0) and (not obj_mask_any)

    if noobj_mask_any:
        logits = predictions[..., 4].reshape(-1)
        n = logits.shape[0]
        lane = 128
        rows_per_block = 512
        block = lane * rows_per_block
        padded = ((n + block - 1) // block) * block
        logits = jnp.pad(logits, (0, padded - n), constant_values=-1e30)
        x2d = logits.reshape(padded // lane, lane)
        grid = padded // block
        out = pl.pallas_call(
            _make_softplus_sum_kernel(0.5 / n),
            grid=(grid,),
            in_specs=[pl.BlockSpec((rows_per_block, lane), lambda i: (i, 0))],
            out_specs=pl.BlockSpec((1, 1), lambda i: (0, 0)),
            out_shape=jax.ShapeDtypeStruct((1, 1), jnp.float32),
        )(x2d)
        return out[0, 0]

    out = _sc_zero()
    return out[0]


# speedup vs baseline: 34.4297x; 34.4297x over previous
import jax
import jax.numpy as jnp
from jax.experimental import pallas as pl
from jax.experimental.pallas import tpu as pltpu


def _zero_scalar_kernel(o_ref):
    o_ref[...] = jnp.zeros((1, 1), jnp.float32)


def _make_softplus_sum_kernel(scale):
    def _softplus_sum_kernel(x_ref, o_ref):
        @pl.when(pl.program_id(0) == 0)
        def _init():
            o_ref[...] = jnp.zeros((1, 1), jnp.float32)

        x = x_ref[...]
        sp = jnp.maximum(x, 0.0) + jnp.log1p(jnp.exp(-jnp.abs(x)))
        o_ref[...] += (jnp.sum(sp) * scale).reshape(1, 1)

    return _softplus_sum_kernel


def kernel(predictions, targets):
    B, A, H, W = predictions.shape
    n_targets = targets.shape[0]
    total_elems = B * A * H * W
    obj_mask_any = (n_targets > 0) and (total_elems > 0)
    noobj_mask_any = (total_elems > 0) and (not obj_mask_any)

    if noobj_mask_any:
        logits = predictions[..., 4].reshape(-1)
        n = logits.shape[0]
        lane = 128
        rows_per_block = 512
        block = lane * rows_per_block
        padded = ((n + block - 1) // block) * block
        logits = jnp.pad(logits, (0, padded - n), constant_values=-1e30)
        x2d = logits.reshape(padded // lane, lane)
        grid = padded // block
        out = pl.pallas_call(
            _make_softplus_sum_kernel(0.5 / n),
            grid=(grid,),
            in_specs=[pl.BlockSpec((rows_per_block, lane), lambda i: (i, 0))],
            out_specs=pl.BlockSpec((1, 1), lambda i: (0, 0)),
            out_shape=jax.ShapeDtypeStruct((1, 1), jnp.float32),
        )(x2d)
        return out[0, 0]

    out = pl.pallas_call(
        _zero_scalar_kernel,
        out_shape=jax.ShapeDtypeStruct((1, 1), jnp.float32),
    )()
    return out[0, 0]
